# baseline (speedup 1.0000x reference)
import numpy as np
import jax
import jax.numpy as jnp
from jax import lax
from jax.experimental import pallas as pl
from jax.experimental.pallas import tpu as pltpu

_T = 500

_ROT_A = (13, 15, 26, 6)
_ROT_B = (17, 29, 16, 24)
_M32 = 0xFFFFFFFF


def _tf_py(k1, k2, x0, x1):
    ks = (k1, k2, k1 ^ k2 ^ 0x1BD11BDA)

    def rot(x, r):
        return ((x << r) | (x >> (32 - r))) & _M32

    def rnds(x0, x1, rots):
        for r in rots:
            x0 = (x0 + x1) & _M32
            x1 = x0 ^ rot(x1, r)
        return x0, x1

    x0 = (x0 + ks[0]) & _M32
    x1 = (x1 + ks[1]) & _M32
    x0, x1 = rnds(x0, x1, _ROT_A)
    x0 = (x0 + ks[1]) & _M32
    x1 = (x1 + ks[2] + 1) & _M32
    x0, x1 = rnds(x0, x1, _ROT_B)
    x0 = (x0 + ks[2]) & _M32
    x1 = (x1 + ks[0] + 2) & _M32
    x0, x1 = rnds(x0, x1, _ROT_A)
    x0 = (x0 + ks[0]) & _M32
    x1 = (x1 + ks[1] + 3) & _M32
    x0, x1 = rnds(x0, x1, _ROT_B)
    x0 = (x0 + ks[1]) & _M32
    x1 = (x1 + ks[2] + 4) & _M32
    x0, x1 = rnds(x0, x1, _ROT_A)
    x0 = (x0 + ks[2]) & _M32
    x1 = (x1 + ks[0] + 5) & _M32
    return x0, x1


_KG = _tf_py(0, 42, 0, 0)
_KU = _tf_py(0, 42, 0, 1)


def _rotl(x, r):
    return lax.shift_left(x, np.uint32(r)) | lax.shift_right_logical(
        x, np.uint32(32 - r))


def _tf(k1, k2, x0, x1):
    k1 = np.uint32(k1)
    k2 = np.uint32(k2)
    ks2 = np.uint32(k1 ^ k2 ^ np.uint32(0x1BD11BDA))

    def rnds(x0, x1, rots):
        for r in rots:
            x0 = x0 + x1
            x1 = x0 ^ _rotl(x1, r)
        return x0, x1

    x0 = x0 + k1
    x1 = x1 + k2
    x0, x1 = rnds(x0, x1, _ROT_A)
    x0 = x0 + k2
    x1 = x1 + np.uint32(ks2 + np.uint32(1))
    x0, x1 = rnds(x0, x1, _ROT_B)
    x0 = x0 + ks2
    x1 = x1 + np.uint32(k1 + np.uint32(2))
    x0, x1 = rnds(x0, x1, _ROT_A)
    x0 = x0 + k1
    x1 = x1 + np.uint32(k2 + np.uint32(3))
    x0, x1 = rnds(x0, x1, _ROT_B)
    x0 = x0 + k2
    x1 = x1 + np.uint32(ks2 + np.uint32(4))
    x0, x1 = rnds(x0, x1, _ROT_A)
    x0 = x0 + ks2
    x1 = x1 + np.uint32(k1 + np.uint32(5))
    return x0, x1


def _tf_var(k1, k2, x0, x1):
    ks2 = k1 ^ k2 ^ np.uint32(0x1BD11BDA)

    def rnds(x0, x1, rots):
        for r in rots:
            x0 = x0 + x1
            x1 = x0 ^ _rotl(x1, r)
        return x0, x1

    x0 = x0 + k1
    x1 = x1 + k2
    x0, x1 = rnds(x0, x1, _ROT_A)
    x0 = x0 + k2
    x1 = x1 + ks2 + np.uint32(1)
    x0, x1 = rnds(x0, x1, _ROT_B)
    x0 = x0 + ks2
    x1 = x1 + k1 + np.uint32(2)
    x0, x1 = rnds(x0, x1, _ROT_A)
    x0 = x0 + k1
    x1 = x1 + k2 + np.uint32(3)
    x0, x1 = rnds(x0, x1, _ROT_B)
    x0 = x0 + k2
    x1 = x1 + ks2 + np.uint32(4)
    x0, x1 = rnds(x0, x1, _ROT_A)
    x0 = x0 + ks2
    x1 = x1 + k1 + np.uint32(5)
    return x0, x1


def _tf_const(k1, k2, c):
    z = jnp.zeros_like(k1)
    return _tf_var(k1, k2, z, z + np.uint32(c))


def _bits_to_unit(bits):
    fb = lax.shift_right_logical(bits, np.uint32(9)) | np.uint32(0x3F800000)
    return lax.bitcast_convert_type(fb, jnp.float32) - np.float32(1.0)


_NLO = np.nextafter(np.float32(-1.0), np.float32(0.0), dtype=np.float32)
_NSPAN = np.float32(np.float32(1.0) - _NLO)
_SQRT2 = np.array(np.sqrt(2), np.float32)
_THIRD = np.float32(1.0 / 3.0)
_SQUEEZE = np.float32(0.0331)


def _gamma_tile(alpha, flat_idx):
    one = np.float32(1.0)

    ka, kb = _tf(_KG[0], _KG[1], jnp.zeros_like(flat_idx), flat_idx)

    boost_mask = alpha >= one
    alpha_orig = alpha
    alphap = jnp.where(boost_mask, alpha, alpha + one)
    d = alphap - _THIRD
    c = _THIRD * lax.rsqrt(d)

    sk_a, sk_b = _tf_const(ka, kb, 1)
    ka, kb = _tf_const(ka, kb, 0)

    X = jnp.zeros_like(alpha)
    V = jnp.ones_like(alpha)
    acc = jnp.zeros(alpha.shape, dtype=jnp.int32)

    def outer_cond(st):
        return jnp.any(st[0] == 0)

    def outer_body(st):
        acc, ka, kb, X, V = st
        accept = acc != 0
        nka, nkb = _tf_const(ka, kb, 0)
        xka, xkb = _tf_const(ka, kb, 1)
        uka, ukb = _tf_const(ka, kb, 2)

        def inner_cond(ist):
            return jnp.any(~accept & (ist[1] <= np.float32(0.0)))

        def inner_body(ist):
            x, v, a, b = ist
            act = ~accept & (v <= np.float32(0.0))
            na, nb = _tf_const(a, b, 0)
            sa, sb = _tf_const(a, b, 1)
            b1, b2 = _tf_const(sa, sb, 0)
            f = _bits_to_unit(b1 ^ b2)
            un = jnp.maximum(_NLO, f * _NSPAN + _NLO)
            xn = _SQRT2 * lax.erf_inv(un)
            vn = one + xn * c
            return (jnp.where(act, xn, x), jnp.where(act, vn, v),
                    jnp.where(act, na, a), jnp.where(act, nb, b))

        x, v, _, _ = lax.while_loop(
            inner_cond, inner_body,
            (jnp.zeros_like(X), jnp.full_like(X, -1.0), xka, xkb))

        Xn = x * x
        Vn = (v * v) * v
        ub1, ub2 = _tf_const(uka, ukb, 0)
        U = _bits_to_unit(ub1 ^ ub2)
        reject = (U >= one - _SQUEEZE * (Xn * Xn)) & \
                 (lax.log(U) >= Xn * np.float32(0.5) +
                  d * ((one - Vn) + lax.log(Vn)))
        X = jnp.where(accept, X, Xn)
        V = jnp.where(accept, V, Vn)
        ka2 = jnp.where(accept, ka, nka)
        kb2 = jnp.where(accept, kb, nkb)
        acc2 = jnp.where(accept | ~reject, jnp.int32(1), jnp.int32(0))
        return (acc2, ka2, kb2, X, V)

    acc, ka, kb, X, V = lax.while_loop(
        outer_cond, outer_body, (acc, ka, kb, X, V))

    bb1, bb2 = _tf_const(sk_a, sk_b, 0)
    samples = one - _bits_to_unit(bb1 ^ bb2)
    boost = jnp.where(boost_mask, one, lax.pow(samples, one / alpha_orig))
    return (d * V) * boost


def _gamma_kernel_body(a_ref, g_ref):
    pid = pl.program_id(0)
    R, C = a_ref.shape
    flat = ((pid * R + lax.broadcasted_iota(jnp.int32, (R, C), 0)) * C
            + lax.broadcasted_iota(jnp.int32, (R, C), 1)).astype(jnp.uint32)
    g_ref[...] = _gamma_tile(a_ref[...], flat)


def _u_kernel_body(B, u_ref):
    R, C = u_ref.shape
    rowflat = lax.broadcasted_iota(jnp.int32, (R, C), 0) * _T
    col = lax.broadcasted_iota(jnp.int32, (R, C), 1)
    flat = (rowflat + col).astype(jnp.uint32)
    b1, b2 = _tf(_KU[0], _KU[1], jnp.zeros_like(flat), flat)
    u_ref[...] = _bits_to_unit(b1 ^ b2)


def _sample_gamma(alpha):
    B, V = alpha.shape
    N = B * V
    R, C = 40, 128
    assert N % (R * C) == 0
    rows = N // C
    af = alpha.reshape(rows, C)
    gf = pl.pallas_call(
        _gamma_kernel_body,
        grid=(rows // R,),
        in_specs=[pl.BlockSpec((R, C), lambda i: (i, 0))],
        out_specs=pl.BlockSpec((R, C), lambda i: (i, 0)),
        out_shape=jax.ShapeDtypeStruct((rows, C), jnp.float32),
        compiler_params=pltpu.CompilerParams(
            dimension_semantics=("parallel",)),
    )(af)
    return gf.reshape(B, V)


def _sample_u(B):
    return pl.pallas_call(
        lambda u: _u_kernel_body(B, u),
        out_shape=jax.ShapeDtypeStruct((B, 512), jnp.float32),
    )()


_UPAD = 512


def _make_sc_multinomial(B, V):
    import functools
    from jax.experimental.pallas import tpu_sc as plsc

    info = plsc.get_sparse_core_info()
    NC, NS = info.num_cores, info.num_subcores
    NW = NC * NS
    assert B % NW == 0 and V % 16 == 0 and _UPAD % 16 == 0
    rows_per = B // NW
    start_step = 1 << ((V - 1).bit_length() - 1)
    mesh = plsc.VectorSubcoreMesh(core_axis_name="c", subcore_axis_name="s")

    @functools.partial(
        pl.kernel, mesh=mesh,
        out_type=jax.ShapeDtypeStruct((B, V), jnp.float32),
        compiler_params=pltpu.CompilerParams(needs_layout_passes=False),
        scratch_types=[
            pltpu.VMEM((V,), jnp.float32),
            pltpu.VMEM((_UPAD,), jnp.float32),
            pltpu.VMEM((_UPAD,), jnp.int32),
        ],
    )
    def sc_kernel(cdf_hbm, u_hbm, out_hbm, cbuf, ubuf, catbuf):
        wid = lax.axis_index("s") * NC + lax.axis_index("c")
        lane = lax.iota(jnp.int32, 16)
        ones = jnp.ones((16,), jnp.float32)
        for k in range(rows_per):
            r = wid * rows_per + k
            pltpu.sync_copy(cdf_hbm.at[r], cbuf)
            pltpu.sync_copy(u_hbm.at[r], ubuf)

            def search_body(j, carry):
                u = ubuf[pl.ds(j * 16, 16)]
                pos = jnp.zeros((16,), jnp.int32)
                step = start_step
                while step:
                    cand = pos + step
                    probe = jnp.minimum(cand, V) - 1
                    val = plsc.load_gather(cbuf, [probe])
                    take = (cand <= V) & (val < u)
                    pos = jnp.where(take, cand, pos)
                    step >>= 1
                catbuf[pl.ds(j * 16, 16)] = jnp.minimum(pos, V - 1)
                return carry

            lax.fori_loop(0, _UPAD // 16, search_body, 0)

            def zero_body(i, carry):
                cbuf[pl.ds(i * 16, 16)] = jnp.zeros((16,), jnp.float32)
                return carry

            lax.fori_loop(0, V // 16, zero_body, 0)

            def scat_body(j, carry):
                cat = catbuf[pl.ds(j * 16, 16)]
                mask = (j * 16 + lane) < _T
                plsc.addupdate_scatter(cbuf, [cat], ones, mask=mask)
                return carry

            lax.fori_loop(0, _UPAD // 16, scat_body, 0)
            pltpu.sync_copy(cbuf, out_hbm.at[r])

    return sc_kernel


def kernel(inpt):
    alpha = inpt
    B, V = alpha.shape
    g = _sample_gamma(alpha)
    u = _sample_u(B)
    denom = jnp.sum(g, axis=-1, keepdims=True)
    p = g / jnp.maximum(denom, 1e-30)
    cdf = jnp.cumsum(p, axis=-1)
    counts = _make_sc_multinomial(B, V)(cdf, u)
    return counts.astype(jnp.int32)


# speedup vs baseline: 1.1659x; 1.1659x over previous
import numpy as np
import jax
import jax.numpy as jnp
from jax import lax
from jax.experimental import pallas as pl
from jax.experimental.pallas import tpu as pltpu

_T = 500

_ROT_A = (13, 15, 26, 6)
_ROT_B = (17, 29, 16, 24)
_M32 = 0xFFFFFFFF


def _tf_py(k1, k2, x0, x1):
    ks = (k1, k2, k1 ^ k2 ^ 0x1BD11BDA)

    def rot(x, r):
        return ((x << r) | (x >> (32 - r))) & _M32

    def rnds(x0, x1, rots):
        for r in rots:
            x0 = (x0 + x1) & _M32
            x1 = x0 ^ rot(x1, r)
        return x0, x1

    x0 = (x0 + ks[0]) & _M32
    x1 = (x1 + ks[1]) & _M32
    x0, x1 = rnds(x0, x1, _ROT_A)
    x0 = (x0 + ks[1]) & _M32
    x1 = (x1 + ks[2] + 1) & _M32
    x0, x1 = rnds(x0, x1, _ROT_B)
    x0 = (x0 + ks[2]) & _M32
    x1 = (x1 + ks[0] + 2) & _M32
    x0, x1 = rnds(x0, x1, _ROT_A)
    x0 = (x0 + ks[0]) & _M32
    x1 = (x1 + ks[1] + 3) & _M32
    x0, x1 = rnds(x0, x1, _ROT_B)
    x0 = (x0 + ks[1]) & _M32
    x1 = (x1 + ks[2] + 4) & _M32
    x0, x1 = rnds(x0, x1, _ROT_A)
    x0 = (x0 + ks[2]) & _M32
    x1 = (x1 + ks[0] + 5) & _M32
    return x0, x1


_KG = _tf_py(0, 42, 0, 0)
_KU = _tf_py(0, 42, 0, 1)


def _rotl(x, r):
    return lax.shift_left(x, np.uint32(r)) | lax.shift_right_logical(
        x, np.uint32(32 - r))


def _tf(k1, k2, x0, x1):
    k1 = np.uint32(k1)
    k2 = np.uint32(k2)
    ks2 = np.uint32(k1 ^ k2 ^ np.uint32(0x1BD11BDA))

    def rnds(x0, x1, rots):
        for r in rots:
            x0 = x0 + x1
            x1 = x0 ^ _rotl(x1, r)
        return x0, x1

    x0 = x0 + k1
    x1 = x1 + k2
    x0, x1 = rnds(x0, x1, _ROT_A)
    x0 = x0 + k2
    x1 = x1 + np.uint32(ks2 + np.uint32(1))
    x0, x1 = rnds(x0, x1, _ROT_B)
    x0 = x0 + ks2
    x1 = x1 + np.uint32(k1 + np.uint32(2))
    x0, x1 = rnds(x0, x1, _ROT_A)
    x0 = x0 + k1
    x1 = x1 + np.uint32(k2 + np.uint32(3))
    x0, x1 = rnds(x0, x1, _ROT_B)
    x0 = x0 + k2
    x1 = x1 + np.uint32(ks2 + np.uint32(4))
    x0, x1 = rnds(x0, x1, _ROT_A)
    x0 = x0 + ks2
    x1 = x1 + np.uint32(k1 + np.uint32(5))
    return x0, x1


def _tf_var(k1, k2, x0, x1):
    ks2 = k1 ^ k2 ^ np.uint32(0x1BD11BDA)

    def rnds(x0, x1, rots):
        for r in rots:
            x0 = x0 + x1
            x1 = x0 ^ _rotl(x1, r)
        return x0, x1

    x0 = x0 + k1
    x1 = x1 + k2
    x0, x1 = rnds(x0, x1, _ROT_A)
    x0 = x0 + k2
    x1 = x1 + ks2 + np.uint32(1)
    x0, x1 = rnds(x0, x1, _ROT_B)
    x0 = x0 + ks2
    x1 = x1 + k1 + np.uint32(2)
    x0, x1 = rnds(x0, x1, _ROT_A)
    x0 = x0 + k1
    x1 = x1 + k2 + np.uint32(3)
    x0, x1 = rnds(x0, x1, _ROT_B)
    x0 = x0 + k2
    x1 = x1 + ks2 + np.uint32(4)
    x0, x1 = rnds(x0, x1, _ROT_A)
    x0 = x0 + ks2
    x1 = x1 + k1 + np.uint32(5)
    return x0, x1


def _tf_const(k1, k2, c):
    z = jnp.zeros_like(k1)
    return _tf_var(k1, k2, z, z + np.uint32(c))


def _bits_to_unit(bits):
    fb = lax.shift_right_logical(bits, np.uint32(9)) | np.uint32(0x3F800000)
    return lax.bitcast_convert_type(fb, jnp.float32) - np.float32(1.0)


_NLO = np.nextafter(np.float32(-1.0), np.float32(0.0), dtype=np.float32)
_NSPAN = np.float32(np.float32(1.0) - _NLO)
_SQRT2 = np.array(np.sqrt(2), np.float32)
_THIRD = np.float32(1.0 / 3.0)
_SQUEEZE = np.float32(0.0331)


def _gamma_tile(alpha, flat_idx):
    one = np.float32(1.0)

    ka, kb = _tf(_KG[0], _KG[1], jnp.zeros_like(flat_idx), flat_idx)

    boost_mask = alpha >= one
    alpha_orig = alpha
    alphap = jnp.where(boost_mask, alpha, alpha + one)
    d = alphap - _THIRD
    c = _THIRD * lax.rsqrt(d)

    sk_a, sk_b = _tf_const(ka, kb, 1)
    ka, kb = _tf_const(ka, kb, 0)

    X = jnp.zeros_like(alpha)
    V = jnp.ones_like(alpha)
    acc = jnp.zeros(alpha.shape, dtype=jnp.int32)

    def outer_cond(st):
        return jnp.any(st[0] == 0)

    def outer_body(st):
        acc, ka, kb, X, V = st
        accept = acc != 0
        nka, nkb = _tf_const(ka, kb, 0)
        xka, xkb = _tf_const(ka, kb, 1)
        uka, ukb = _tf_const(ka, kb, 2)

        def inner_cond(ist):
            return jnp.any(~accept & (ist[1] <= np.float32(0.0)))

        def inner_body(ist):
            x, v, a, b = ist
            act = ~accept & (v <= np.float32(0.0))
            na, nb = _tf_const(a, b, 0)
            sa, sb = _tf_const(a, b, 1)
            b1, b2 = _tf_const(sa, sb, 0)
            f = _bits_to_unit(b1 ^ b2)
            un = jnp.maximum(_NLO, f * _NSPAN + _NLO)
            xn = _SQRT2 * lax.erf_inv(un)
            vn = one + xn * c
            return (jnp.where(act, xn, x), jnp.where(act, vn, v),
                    jnp.where(act, na, a), jnp.where(act, nb, b))

        x, v, _, _ = lax.while_loop(
            inner_cond, inner_body,
            (jnp.zeros_like(X), jnp.full_like(X, -1.0), xka, xkb))

        Xn = x * x
        Vn = (v * v) * v
        ub1, ub2 = _tf_const(uka, ukb, 0)
        U = _bits_to_unit(ub1 ^ ub2)
        reject = (U >= one - _SQUEEZE * (Xn * Xn)) & \
                 (lax.log(U) >= Xn * np.float32(0.5) +
                  d * ((one - Vn) + lax.log(Vn)))
        X = jnp.where(accept, X, Xn)
        V = jnp.where(accept, V, Vn)
        ka2 = jnp.where(accept, ka, nka)
        kb2 = jnp.where(accept, kb, nkb)
        acc2 = jnp.where(accept | ~reject, jnp.int32(1), jnp.int32(0))
        return (acc2, ka2, kb2, X, V)

    acc, ka, kb, X, V = lax.while_loop(
        outer_cond, outer_body, (acc, ka, kb, X, V))

    bb1, bb2 = _tf_const(sk_a, sk_b, 0)
    samples = one - _bits_to_unit(bb1 ^ bb2)
    boost = jnp.where(boost_mask, one, lax.pow(samples, one / alpha_orig))
    return (d * V) * boost


def _gamma_kernel_body(a_ref, g_ref):
    pid = pl.program_id(0)
    R, C = a_ref.shape
    flat = ((pid * R + lax.broadcasted_iota(jnp.int32, (R, C), 0)) * C
            + lax.broadcasted_iota(jnp.int32, (R, C), 1)).astype(jnp.uint32)
    g_ref[...] = _gamma_tile(a_ref[...], flat)


def _u_kernel_body(B, u_ref):
    R, C = u_ref.shape
    rowflat = lax.broadcasted_iota(jnp.int32, (R, C), 0) * _T
    col = lax.broadcasted_iota(jnp.int32, (R, C), 1)
    flat = (rowflat + col).astype(jnp.uint32)
    b1, b2 = _tf(_KU[0], _KU[1], jnp.zeros_like(flat), flat)
    u_ref[...] = _bits_to_unit(b1 ^ b2)


def _sample_gamma(alpha):
    B, V = alpha.shape
    N = B * V
    R, C = 40, 256
    assert N % (R * C) == 0
    rows = N // C
    af = alpha.reshape(rows, C)
    gf = pl.pallas_call(
        _gamma_kernel_body,
        grid=(rows // R,),
        in_specs=[pl.BlockSpec((R, C), lambda i: (i, 0))],
        out_specs=pl.BlockSpec((R, C), lambda i: (i, 0)),
        out_shape=jax.ShapeDtypeStruct((rows, C), jnp.float32),
        compiler_params=pltpu.CompilerParams(
            dimension_semantics=("parallel",)),
    )(af)
    return gf.reshape(B, V)


def _sample_u(B):
    return pl.pallas_call(
        lambda u: _u_kernel_body(B, u),
        out_shape=jax.ShapeDtypeStruct((B, 512), jnp.float32),
    )()


_UPAD = 512


def _make_sc_multinomial(B, V):
    import functools
    from jax.experimental.pallas import tpu_sc as plsc

    info = plsc.get_sparse_core_info()
    NC, NS = info.num_cores, info.num_subcores
    NW = NC * NS
    assert B % NW == 0 and V % 16 == 0 and _UPAD % 16 == 0
    rows_per = B // NW
    start_step = 1 << ((V - 1).bit_length() - 1)
    mesh = plsc.VectorSubcoreMesh(core_axis_name="c", subcore_axis_name="s")

    @functools.partial(
        pl.kernel, mesh=mesh,
        out_type=jax.ShapeDtypeStruct((B, V), jnp.float32),
        compiler_params=pltpu.CompilerParams(needs_layout_passes=False),
        scratch_types=[
            pltpu.VMEM((V,), jnp.float32),
            pltpu.VMEM((_UPAD,), jnp.float32),
            pltpu.VMEM((_UPAD,), jnp.int32),
        ],
    )
    def sc_kernel(cdf_hbm, u_hbm, out_hbm, cbuf, ubuf, catbuf):
        wid = lax.axis_index("s") * NC + lax.axis_index("c")
        lane = lax.iota(jnp.int32, 16)
        ones = jnp.ones((16,), jnp.float32)
        for k in range(rows_per):
            r = wid * rows_per + k
            pltpu.sync_copy(cdf_hbm.at[r], cbuf)
            pltpu.sync_copy(u_hbm.at[r], ubuf)

            def search_body(j, carry):
                u = ubuf[pl.ds(j * 16, 16)]
                pos = jnp.zeros((16,), jnp.int32)
                step = start_step
                while step:
                    cand = pos + step
                    probe = jnp.minimum(cand, V) - 1
                    val = plsc.load_gather(cbuf, [probe])
                    take = (cand <= V) & (val < u)
                    pos = jnp.where(take, cand, pos)
                    step >>= 1
                catbuf[pl.ds(j * 16, 16)] = jnp.minimum(pos, V - 1)
                return carry

            lax.fori_loop(0, _UPAD // 16, search_body, 0)

            def zero_body(i, carry):
                cbuf[pl.ds(i * 16, 16)] = jnp.zeros((16,), jnp.float32)
                return carry

            lax.fori_loop(0, V // 16, zero_body, 0)

            def scat_body(j, carry):
                cat = catbuf[pl.ds(j * 16, 16)]
                mask = (j * 16 + lane) < _T
                plsc.addupdate_scatter(cbuf, [cat], ones, mask=mask)
                return carry

            lax.fori_loop(0, _UPAD // 16, scat_body, 0)
            pltpu.sync_copy(cbuf, out_hbm.at[r])

    return sc_kernel


def kernel(inpt):
    alpha = inpt
    B, V = alpha.shape
    g = _sample_gamma(alpha)
    u = _sample_u(B)
    denom = jnp.sum(g, axis=-1, keepdims=True)
    p = g / jnp.maximum(denom, 1e-30)
    cdf = jnp.cumsum(p, axis=-1)
    counts = _make_sc_multinomial(B, V)(cdf, u)
    return counts.astype(jnp.int32)
